# baseline (speedup 1.0000x reference)
import jax
import jax.numpy as jnp
from jax.experimental import pallas as pl
from jax.experimental.pallas import tpu as pltpu

_NUM_LAYERS = 3
_BN_EPS = 1e-5
_GRP_ROWS = 128


def _bn_relu(z, n_nodes, gamma, beta):
    inv_n = 1.0 / n_nodes
    s1 = jnp.sum(z, axis=1, keepdims=True)
    s2 = jnp.sum(z * z, axis=1, keepdims=True)
    mu = s1 * inv_n
    var = s2 * inv_n - mu * mu
    scale = gamma * jax.lax.rsqrt(var + _BN_EPS)
    shift = beta - mu * scale
    return jnp.maximum(z * scale + shift, 0.0)


def _make_kernel(n_grp, b_grp, n_nodes, g_graphs, f_in, hid, num_layers):
    m_grp = b_grp * n_nodes
    g_grp = b_grp * g_graphs

    def body(eps_ref, x_ref, adj_ref, pool_ref, w1_ref, w2_ref, vec_ref,
             node_ref, graph_ref):
        row_g = jax.lax.broadcasted_iota(jnp.int32, (m_grp, m_grp), 0) // n_nodes
        col_g = jax.lax.broadcasted_iota(jnp.int32, (m_grp, m_grp), 1) // n_nodes
        adj_mask = row_g == col_g
        prow_g = jax.lax.broadcasted_iota(jnp.int32, (g_grp, m_grp), 0) // g_graphs
        pcol_g = jax.lax.broadcasted_iota(jnp.int32, (g_grp, m_grp), 1) // n_nodes
        pool_mask = prow_g == pcol_g
        zero = jnp.float32(0.0)

        bds = []
        pbds = []
        for g in range(n_grp):
            a2 = adj_ref[g * m_grp:(g + 1) * m_grp, :]
            rep = pltpu.repeat(a2, b_grp, axis=1)
            bds.append(jnp.where(adj_mask, rep, zero).astype(jnp.bfloat16))
            p2 = pool_ref[g * g_grp:(g + 1) * g_grp, :]
            prep = pltpu.repeat(p2, b_grp, axis=1)
            pbds.append(jnp.where(pool_mask, prep, zero).astype(jnp.bfloat16))

        wave = min(16, n_grp)
        for w0 in range(0, n_grp, wave):
            gids = range(w0, w0 + wave)
            hs = {g: x_ref[g * m_grp:(g + 1) * m_grp, :] for g in gids}
            for i in range(num_layers):
                one_plus_eps = eps_ref[i]
                w1 = w1_ref[i, 0:hs[w0].shape[1], :]
                g1 = vec_ref[4 * i + 0, :].reshape(1, 1, hid)
                be1 = vec_ref[4 * i + 1, :].reshape(1, 1, hid)
                g2 = vec_ref[4 * i + 2, :].reshape(1, 1, hid)
                be2 = vec_ref[4 * i + 3, :].reshape(1, 1, hid)

                msgs = {g: jnp.dot(bds[g], hs[g].astype(jnp.bfloat16),
                                   preferred_element_type=jnp.float32)
                        for g in gids}
                zs = {g: one_plus_eps * hs[g] + msgs[g] for g in gids}
                ys = {g: jnp.dot(zs[g].astype(jnp.bfloat16), w1,
                                 preferred_element_type=jnp.float32)
                      for g in gids}
                ys = {g: _bn_relu(ys[g].reshape(b_grp, n_nodes, hid),
                                  n_nodes, g1, be1) for g in gids}
                ys = {g: jnp.dot(ys[g].reshape(m_grp, hid).astype(jnp.bfloat16),
                                 w2_ref[i], preferred_element_type=jnp.float32)
                      for g in gids}
                hs = {g: _bn_relu(ys[g].reshape(b_grp, n_nodes, hid),
                                  n_nodes, g2, be2).reshape(m_grp, hid)
                      for g in gids}
                gouts = {g: jnp.dot(pbds[g], hs[g].astype(jnp.bfloat16),
                                    preferred_element_type=jnp.float32)
                         for g in gids}
                for g in gids:
                    node_ref[g * m_grp:(g + 1) * m_grp,
                             i * hid:(i + 1) * hid] = hs[g]
                    graph_ref[g * g_grp:(g + 1) * g_grp,
                              i * hid:(i + 1) * hid] = gouts[g]

    return body


def kernel(x, adj, pool,
           eps_0, w1_0, b1_0, g1_0, be1_0, w2_0, b2_0, g2_0, be2_0,
           eps_1, w1_1, b1_1, g1_1, be1_1, w2_1, b2_1, g2_1, be2_1,
           eps_2, w1_2, b1_2, g1_2, be1_2, w2_2, b2_2, g2_2, be2_2):
    B, N, F = x.shape
    G = pool.shape[1]
    hid = w2_0.shape[1]
    L = _NUM_LAYERS
    params = [
        (eps_0, w1_0, g1_0, be1_0, w2_0, g2_0, be2_0),
        (eps_1, w1_1, g1_1, be1_1, w2_1, g2_1, be2_1),
        (eps_2, w1_2, g1_2, be1_2, w2_2, g2_2, be2_2),
    ]

    b_grp = max(1, _GRP_ROWS // N)
    total_rows = B * N
    rows_blk = min(32 * b_grp * N, total_rows)
    n_grp = rows_blk // (b_grp * N)
    nb = total_rows // rows_blk
    grows_blk = rows_blk // N * G

    x_flat = x.reshape(total_rows, F)
    adj_flat = adj.reshape(total_rows, N)
    pool_flat = pool.reshape(B * G, N)

    def pad_w(w):
        return jnp.pad(w, ((0, hid - w.shape[0]), (0, 0)))

    w1s = jnp.stack([pad_w(p[1]) for p in params]).astype(jnp.bfloat16)
    w2s = jnp.stack([p[4] for p in params]).astype(jnp.bfloat16)
    vec = jnp.concatenate(
        [jnp.concatenate([p[2], p[3], p[5], p[6]], axis=0) for p in params],
        axis=0)
    eps_arr = jnp.stack(
        [1.0 + p[0].reshape(()).astype(jnp.float32) for p in params])

    body = _make_kernel(n_grp, b_grp, N, G, F, hid, L)

    node_flat, graph_flat = pl.pallas_call(
        body,
        out_shape=(
            jax.ShapeDtypeStruct((total_rows, L * hid), jnp.float32),
            jax.ShapeDtypeStruct((B * G, L * hid), jnp.float32),
        ),
        grid_spec=pltpu.PrefetchScalarGridSpec(
            num_scalar_prefetch=1,
            grid=(nb,),
            in_specs=[
                pl.BlockSpec((rows_blk, F), lambda b, eps: (b, 0)),
                pl.BlockSpec((rows_blk, N), lambda b, eps: (b, 0)),
                pl.BlockSpec((grows_blk, N), lambda b, eps: (b, 0)),
                pl.BlockSpec((L, hid, hid), lambda b, eps: (0, 0, 0)),
                pl.BlockSpec((L, hid, hid), lambda b, eps: (0, 0, 0)),
                pl.BlockSpec((4 * L, hid), lambda b, eps: (0, 0)),
            ],
            out_specs=(
                pl.BlockSpec((rows_blk, L * hid), lambda b, eps: (b, 0)),
                pl.BlockSpec((grows_blk, L * hid), lambda b, eps: (b, 0)),
            ),
        ),
        compiler_params=pltpu.CompilerParams(
            dimension_semantics=("parallel",),
            vmem_limit_bytes=64 * 1024 * 1024,
        ),
    )(eps_arr, x_flat, adj_flat, pool_flat, w1s, w2s, vec)

    node_rep = node_flat.reshape(B, N, L * hid)
    graph_rep = graph_flat.reshape(B, G, L * hid)
    return graph_rep, node_rep


# speedup vs baseline: 1.0214x; 1.0214x over previous
import jax
import jax.numpy as jnp
from jax.experimental import pallas as pl
from jax.experimental.pallas import tpu as pltpu

_NUM_LAYERS = 3
_BN_EPS = 1e-5
_GRP_ROWS = 128


def _bn_relu(z, n_nodes, gamma, beta):
    inv_n = 1.0 / n_nodes
    s1 = jnp.sum(z, axis=1, keepdims=True)
    s2 = jnp.sum(z * z, axis=1, keepdims=True)
    mu = s1 * inv_n
    var = s2 * inv_n - mu * mu
    scale = gamma * jax.lax.rsqrt(var + _BN_EPS)
    shift = beta - mu * scale
    return jnp.maximum(
        z.astype(jnp.bfloat16) * scale.astype(jnp.bfloat16)
        + shift.astype(jnp.bfloat16), jnp.bfloat16(0.0))


def _make_kernel(n_grp, b_grp, n_nodes, g_graphs, f_in, hid, num_layers):
    m_grp = b_grp * n_nodes
    g_grp = b_grp * g_graphs

    def body(eps_ref, x_ref, adj_ref, pool_ref, w1_ref, w2_ref, vec_ref,
             node_ref, graph_ref):
        row_g = jax.lax.broadcasted_iota(jnp.int32, (m_grp, m_grp), 0) // n_nodes
        col_g = jax.lax.broadcasted_iota(jnp.int32, (m_grp, m_grp), 1) // n_nodes
        adj_mask = row_g == col_g
        prow_g = jax.lax.broadcasted_iota(jnp.int32, (g_grp, m_grp), 0) // g_graphs
        pcol_g = jax.lax.broadcasted_iota(jnp.int32, (g_grp, m_grp), 1) // n_nodes
        pool_mask = prow_g == pcol_g
        zero = jnp.float32(0.0)

        bds = []
        pbds = []
        for g in range(n_grp):
            a2 = adj_ref[g * m_grp:(g + 1) * m_grp, :]
            rep = pltpu.repeat(a2, b_grp, axis=1)
            bds.append(jnp.where(adj_mask, rep, zero).astype(jnp.bfloat16))
            p2 = pool_ref[g * g_grp:(g + 1) * g_grp, :]
            prep = pltpu.repeat(p2, b_grp, axis=1)
            pbds.append(jnp.where(pool_mask, prep, zero).astype(jnp.bfloat16))

        gids = range(n_grp)
        hfs = {g: x_ref[g * m_grp:(g + 1) * m_grp, :] for g in gids}
        hbs = {g: hfs[g].astype(jnp.bfloat16) for g in gids}
        for i in range(num_layers):
            one_plus_eps = eps_ref[i]
            w1 = w1_ref[i, 0:hfs[0].shape[1], :]
            g1 = vec_ref[4 * i + 0, :].reshape(1, 1, hid)
            be1 = vec_ref[4 * i + 1, :].reshape(1, 1, hid)
            g2 = vec_ref[4 * i + 2, :].reshape(1, 1, hid)
            be2 = vec_ref[4 * i + 3, :].reshape(1, 1, hid)

            msgs = {g: jnp.dot(bds[g], hbs[g],
                               preferred_element_type=jnp.float32)
                    for g in gids}
            zs = {g: one_plus_eps * hfs[g] + msgs[g] for g in gids}
            ys = {g: jnp.dot(zs[g].astype(jnp.bfloat16), w1,
                             preferred_element_type=jnp.float32)
                  for g in gids}
            ys = {g: _bn_relu(ys[g].reshape(b_grp, n_nodes, hid),
                              n_nodes, g1, be1) for g in gids}
            ys = {g: jnp.dot(ys[g].reshape(m_grp, hid), w2_ref[i],
                             preferred_element_type=jnp.float32)
                  for g in gids}
            hbs = {g: _bn_relu(ys[g].reshape(b_grp, n_nodes, hid),
                               n_nodes, g2, be2).reshape(m_grp, hid)
                   for g in gids}
            hfs = {g: hbs[g].astype(jnp.float32) for g in gids}
            gouts = {g: jnp.dot(pbds[g], hbs[g],
                                preferred_element_type=jnp.float32)
                     for g in gids}
            for g in gids:
                node_ref[g * m_grp:(g + 1) * m_grp,
                         i * hid:(i + 1) * hid] = hfs[g]
                graph_ref[g * g_grp:(g + 1) * g_grp,
                          i * hid:(i + 1) * hid] = gouts[g]

    return body


def kernel(x, adj, pool,
           eps_0, w1_0, b1_0, g1_0, be1_0, w2_0, b2_0, g2_0, be2_0,
           eps_1, w1_1, b1_1, g1_1, be1_1, w2_1, b2_1, g2_1, be2_1,
           eps_2, w1_2, b1_2, g1_2, be1_2, w2_2, b2_2, g2_2, be2_2):
    B, N, F = x.shape
    G = pool.shape[1]
    hid = w2_0.shape[1]
    L = _NUM_LAYERS
    params = [
        (eps_0, w1_0, g1_0, be1_0, w2_0, g2_0, be2_0),
        (eps_1, w1_1, g1_1, be1_1, w2_1, g2_1, be2_1),
        (eps_2, w1_2, g1_2, be1_2, w2_2, g2_2, be2_2),
    ]

    b_grp = max(1, _GRP_ROWS // N)
    total_rows = B * N
    rows_blk = min(32 * b_grp * N, total_rows)
    n_grp = rows_blk // (b_grp * N)
    nb = total_rows // rows_blk
    grows_blk = rows_blk // N * G

    x_flat = x.reshape(total_rows, F)
    adj_flat = adj.reshape(total_rows, N)
    pool_flat = pool.reshape(B * G, N)

    def pad_w(w):
        return jnp.pad(w, ((0, hid - w.shape[0]), (0, 0)))

    w1s = jnp.stack([pad_w(p[1]) for p in params]).astype(jnp.bfloat16)
    w2s = jnp.stack([p[4] for p in params]).astype(jnp.bfloat16)
    vec = jnp.concatenate(
        [jnp.concatenate([p[2], p[3], p[5], p[6]], axis=0) for p in params],
        axis=0)
    eps_arr = jnp.stack(
        [1.0 + p[0].reshape(()).astype(jnp.float32) for p in params])

    body = _make_kernel(n_grp, b_grp, N, G, F, hid, L)

    node_flat, graph_flat = pl.pallas_call(
        body,
        out_shape=(
            jax.ShapeDtypeStruct((total_rows, L * hid), jnp.float32),
            jax.ShapeDtypeStruct((B * G, L * hid), jnp.float32),
        ),
        grid_spec=pltpu.PrefetchScalarGridSpec(
            num_scalar_prefetch=1,
            grid=(nb,),
            in_specs=[
                pl.BlockSpec((rows_blk, F), lambda b, eps: (b, 0)),
                pl.BlockSpec((rows_blk, N), lambda b, eps: (b, 0)),
                pl.BlockSpec((grows_blk, N), lambda b, eps: (b, 0)),
                pl.BlockSpec((L, hid, hid), lambda b, eps: (0, 0, 0)),
                pl.BlockSpec((L, hid, hid), lambda b, eps: (0, 0, 0)),
                pl.BlockSpec((4 * L, hid), lambda b, eps: (0, 0)),
            ],
            out_specs=(
                pl.BlockSpec((rows_blk, L * hid), lambda b, eps: (b, 0)),
                pl.BlockSpec((grows_blk, L * hid), lambda b, eps: (b, 0)),
            ),
        ),
        compiler_params=pltpu.CompilerParams(
            dimension_semantics=("parallel",),
            vmem_limit_bytes=64 * 1024 * 1024,
        ),
    )(eps_arr, x_flat, adj_flat, pool_flat, w1s, w2s, vec)

    node_rep = node_flat.reshape(B, N, L * hid)
    graph_rep = graph_flat.reshape(B, G, L * hid)
    return graph_rep, node_rep
